# baseline (speedup 1.0000x reference)
import functools

import jax
import jax.numpy as jnp
from jax import lax
from jax.experimental import pallas as pl
from jax.experimental.pallas import tpu as pltpu
from jax.experimental.pallas import tpu_sc as plsc

N = 10000
NPAD = 10240
E = 320000
NC, NS = 2, 16
NW = NC * NS
CH = 128
NCH = -(-E // (NW * CH))
EPT = NCH * CH
EPAD = EPT * NW

_MESH = plsc.VectorSubcoreMesh(
    core_axis_name="c", subcore_axis_name="s", num_cores=NC, num_subcores=NS
)


@functools.partial(
    pl.kernel,
    out_type=jax.ShapeDtypeStruct((NC, NPAD), jnp.float32),
    mesh=_MESH,
    scratch_types=[
        pltpu.VMEM((NCH, CH), jnp.int32),
        pltpu.VMEM((CH,), jnp.float32),
        pltpu.VMEM_SHARED((NPAD,), jnp.float32),
    ],
)
def _deg_kernel(dst_hbm, ones_hbm, zeros_hbm, degp_hbm, dstv, onesv, deg_sp):
    c = lax.axis_index("c")
    s = lax.axis_index("s")
    w = c * NS + s
    pltpu.sync_copy(dst_hbm.at[w], dstv)
    pltpu.sync_copy(ones_hbm, onesv)

    @pl.when(s == 0)
    def _():
        pltpu.sync_copy(zeros_hbm, deg_sp)

    plsc.subcore_barrier()

    def step(j, carry):
        pltpu.sync_copy(onesv, deg_sp.at[dstv.at[j]], add=True)
        return carry

    lax.fori_loop(0, NCH, step, 0)
    plsc.subcore_barrier()
    rps = NPAD // NS
    pltpu.sync_copy(deg_sp.at[pl.ds(s * rps, rps)], degp_hbm.at[c, pl.ds(s * rps, rps)])


def _make_edge_pass(F):
    scratch = [
        pltpu.VMEM((NCH, CH), jnp.int32),
        pltpu.VMEM((NCH, CH), jnp.int32),
    ]
    scratch += [pltpu.VMEM((CH,), jnp.float32) for _ in range(F)]
    scratch += [pltpu.VMEM_SHARED((NPAD,), jnp.float32) for _ in range(2 * F)]
    scratch += [pltpu.SemaphoreType.DMA]

    @functools.partial(
        pl.kernel,
        out_type=[jax.ShapeDtypeStruct((NPAD,), jnp.float32)] * (NC * F),
        mesh=_MESH,
        scratch_types=scratch,
    )
    def _edge_kernel(src_hbm, dst_hbm, *rest):
        g = rest[0:F]
        zeros_hbm = rest[F]
        outs = rest[F + 1:F + 1 + NC * F]
        scr = rest[F + 1 + NC * F:]
        srcv, dstv = scr[0], scr[1]
        rows = scr[2:2 + F]
        tbl = scr[2 + F:2 + 2 * F]
        acc = scr[2 + 2 * F:2 + 3 * F]
        sem = scr[-1]
        c = lax.axis_index("c")
        s = lax.axis_index("s")
        w = c * NS + s
        pltpu.sync_copy(src_hbm.at[w], srcv)
        pltpu.sync_copy(dst_hbm.at[w], dstv)

        for f in range(F):
            @pl.when(s == f)
            def _(f=f):
                pltpu.sync_copy(g[f], tbl[f])

            @pl.when(jnp.logical_and(s == F + f, c == 0))
            def _(f=f):
                pltpu.sync_copy(g[f], acc[f])

            @pl.when(jnp.logical_and(s == F + f, c == 1))
            def _(f=f):
                pltpu.sync_copy(zeros_hbm, acc[f])

        plsc.subcore_barrier()

        def step(j, carry):
            sidx = srcv.at[j]
            didx = dstv.at[j]
            cps = [pltpu.async_copy(tbl[f].at[sidx], rows[f], sem)
                   for f in range(F)]
            for cp in cps:
                cp.wait()
            cps = [pltpu.async_copy(rows[f], acc[f].at[didx], sem,
                                    add=True) for f in range(F)]
            for cp in cps:
                cp.wait()
            return carry

        lax.fori_loop(0, NCH, step, 0)
        plsc.subcore_barrier()

        for cc in range(NC):
            for f in range(F):
                @pl.when(jnp.logical_and(c == cc, s == f))
                def _(cc=cc, f=f):
                    pltpu.sync_copy(acc[f], outs[cc * F + f])

    return _edge_kernel


_edge_pass_8 = _make_edge_pass(8)

RPS = NPAD // NS


@functools.partial(
    pl.kernel,
    out_type=[jax.ShapeDtypeStruct((NPAD,), jnp.float32)] * (NC * 2),
    mesh=_MESH,
    scratch_types=[
        pltpu.VMEM((NCH, CH), jnp.int32),
        pltpu.VMEM((NCH, CH), jnp.int32),
        pltpu.VMEM((CH,), jnp.float32),
        pltpu.VMEM((CH,), jnp.float32),
        pltpu.VMEM((RPS,), jnp.float32),
        pltpu.VMEM((RPS,), jnp.float32),
        pltpu.VMEM((RPS,), jnp.float32),
        pltpu.VMEM((RPS,), jnp.float32),
        pltpu.VMEM((RPS,), jnp.float32),
        pltpu.VMEM((8, 16), jnp.float32),
        pltpu.VMEM((8, 16), jnp.float32),
        pltpu.VMEM((8, 16), jnp.float32),
        pltpu.VMEM_SHARED((NPAD,), jnp.float32),
        pltpu.VMEM_SHARED((NPAD,), jnp.float32),
        pltpu.VMEM_SHARED((NPAD,), jnp.float32),
        pltpu.VMEM_SHARED((NPAD,), jnp.float32),
        pltpu.SemaphoreType.DMA,
    ],
)
def _edge2_fused(src_hbm, dst_hbm, *rest):
    p = rest[0:16]
    dis_hbm, b1_hbm, w20_hbm, w21_hbm, zeros_hbm = rest[16:21]
    outs = rest[21:25]
    (srcv, dstv, row0, row1, p0v, p1v, disv, g0v, g1v,
     b1v, w20v, w21v, tbl0, tbl1, acc0, acc1, sem) = rest[25:]
    c = lax.axis_index("c")
    s = lax.axis_index("s")
    w = c * NS + s
    pltpu.sync_copy(src_hbm.at[w], srcv)
    pltpu.sync_copy(dst_hbm.at[w], dstv)
    sl_h = pl.ds(s * RPS, RPS)
    pltpu.sync_copy(dis_hbm.at[sl_h], disv)
    pltpu.sync_copy(b1_hbm, b1v)
    pltpu.sync_copy(w20_hbm, w20v)
    pltpu.sync_copy(w21_hbm, w21v)

    for f in range(8):
        pltpu.sync_copy(p[f].at[sl_h], p0v)
        pltpu.sync_copy(p[8 + f].at[sl_h], p1v)

        def body(b, carry, f=f):
            slb = pl.ds(b * 16, 16)
            z = disv[slb] * (p0v[slb] + p1v[slb]) + b1v[f]
            t = 1.0 - 2.0 / (jnp.exp(2.0 * z) + 1.0)
            if f == 0:
                g0v[slb] = w20v[f] * t
                g1v[slb] = w21v[f] * t
            else:
                g0v[slb] = g0v[slb] + w20v[f] * t
                g1v[slb] = g1v[slb] + w21v[f] * t
            return carry

        lax.fori_loop(0, RPS // 16, body, 0)

    def scale(b, carry):
        slb = pl.ds(b * 16, 16)
        g0v[slb] = disv[slb] * g0v[slb]
        g1v[slb] = disv[slb] * g1v[slb]
        return carry

    lax.fori_loop(0, RPS // 16, scale, 0)
    pltpu.sync_copy(g0v, tbl0.at[sl_h])
    pltpu.sync_copy(g1v, tbl1.at[sl_h])

    @pl.when(c == 0)
    def _():
        pltpu.sync_copy(g0v, acc0.at[sl_h])
        pltpu.sync_copy(g1v, acc1.at[sl_h])

    @pl.when(c == 1)
    def _():
        pltpu.sync_copy(zeros_hbm.at[sl_h], acc0.at[sl_h])
        pltpu.sync_copy(zeros_hbm.at[sl_h], acc1.at[sl_h])

    plsc.subcore_barrier()

    def step(j, carry):
        sidx = srcv.at[j]
        didx = dstv.at[j]
        cp0 = pltpu.async_copy(tbl0.at[sidx], row0, sem)
        cp1 = pltpu.async_copy(tbl1.at[sidx], row1, sem)
        cp0.wait()
        cp1.wait()
        cp0 = pltpu.async_copy(row0, acc0.at[didx], sem, add=True)
        cp1 = pltpu.async_copy(row1, acc1.at[didx], sem, add=True)
        cp0.wait()
        cp1.wait()
        return carry

    lax.fori_loop(0, NCH, step, 0)
    plsc.subcore_barrier()

    accs = [acc0, acc1]
    for cc in range(NC):
        for f in range(2):
            @pl.when(jnp.logical_and(c == cc, s == f))
            def _(cc=cc, f=f):
                pltpu.sync_copy(accs[f], outs[cc * 2 + f])


_DN = (((0,), (1,)), ((), ()))
_DF = (((0,), (0,)), ((), ()))


def _tc_mm_body(x_ref, w1_ref, h_ref):
    h_ref[...] = lax.dot_general(w1_ref[...], x_ref[...], _DN,
                                 preferred_element_type=jnp.float32)


def _tc_scale_body(h_ref, degp_ref, g_ref, dis_ref):
    dp = degp_ref[...]
    deg = dp[0:1, :] + dp[1:2, :] + 1.0
    dis = lax.rsqrt(deg)
    g_ref[...] = dis * h_ref[...]
    dis_ref[...] = dis


def _tc_mid_body(accp_ref, dis_ref, b1_ref, w2_ref, g2_ref):
    ap = accp_ref[...]
    dis = dis_ref[...]
    h = jnp.tanh(dis * (ap[0] + ap[1]) + b1_ref[...])
    g2_ref[...] = dis * lax.dot_general(w2_ref[...], h, _DF,
                                        preferred_element_type=jnp.float32)


def _tc_post_body(accp_ref, dis_ref, b2_ref, wc_ref, bc_ref, out_ref):
    ap = accp_ref[...]
    emb = jnp.tanh(dis_ref[...] * (ap[0] + ap[1]) + b2_ref[...])
    z = lax.dot_general(wc_ref[...], emb, _DF,
                        preferred_element_type=jnp.float32) + bc_ref[...]
    out_ref[...] = jax.nn.sigmoid(z)


_tc_mm = pl.pallas_call(
    _tc_mm_body,
    out_shape=jax.ShapeDtypeStruct((8, NPAD), jnp.float32),
)

_tc_scale = pl.pallas_call(
    _tc_scale_body,
    out_shape=(
        jax.ShapeDtypeStruct((8, NPAD), jnp.float32),
        jax.ShapeDtypeStruct((1, NPAD), jnp.float32),
    ),
)

_tc_mid = pl.pallas_call(
    _tc_mid_body,
    out_shape=jax.ShapeDtypeStruct((2, NPAD), jnp.float32),
)

_tc_post = pl.pallas_call(
    _tc_post_body,
    out_shape=jax.ShapeDtypeStruct((1, NPAD), jnp.float32),
)


def kernel(x, edge_index, W1, b1, W2, b2, Wc, bc):
    ei = edge_index.astype(jnp.int32)
    pad = jnp.full((EPAD - E,), NPAD - 1, jnp.int32)
    srcf = jnp.concatenate([ei[0], pad])
    dstf = jnp.concatenate([ei[1], pad])
    srcr = srcf.reshape(NW, NCH, CH)
    dstr = dstf.reshape(NW, NCH, CH)
    x_pad = jnp.pad(x, ((0, NPAD - N), (0, 0)))
    ones_ch = jnp.ones((CH,), jnp.float32)
    zeros1 = jnp.zeros((NPAD,), jnp.float32)

    h1 = _tc_mm(x_pad, W1)
    degp = _deg_kernel(dstr, ones_ch, zeros1)
    g1, dis = _tc_scale(h1, degp)
    outs1 = _edge_pass_8(srcr, dstr, *[g1[f] for f in range(8)], zeros1)
    b1b = jnp.broadcast_to(b1.reshape(8, 1), (8, 16))
    w20b = jnp.broadcast_to(W2[:, 0].reshape(8, 1), (8, 16))
    w21b = jnp.broadcast_to(W2[:, 1].reshape(8, 1), (8, 16))
    outs2 = _edge2_fused(srcr, dstr, *outs1, dis.reshape(NPAD), b1b, w20b,
                         w21b, zeros1)
    accp2 = jnp.stack(outs2).reshape(NC, 2, NPAD)
    out = _tc_post(accp2, dis, b2.reshape(2, 1), Wc, bc.reshape(1, 1))
    return out.reshape(NPAD, 1)[:N]


# speedup vs baseline: 1.0810x; 1.0810x over previous
import functools

import jax
import jax.numpy as jnp
from jax import lax
from jax.experimental import pallas as pl
from jax.experimental.pallas import tpu as pltpu
from jax.experimental.pallas import tpu_sc as plsc

N = 10000
NPAD = 10240
E = 320000
NC, NS = 2, 16
NW = NC * NS
CH = 128
NCH = -(-E // (NW * CH))
EPT = NCH * CH
EPAD = EPT * NW

_MESH = plsc.VectorSubcoreMesh(
    core_axis_name="c", subcore_axis_name="s", num_cores=NC, num_subcores=NS
)


@functools.partial(
    pl.kernel,
    out_type=jax.ShapeDtypeStruct((NC, NPAD), jnp.float32),
    mesh=_MESH,
    scratch_types=[
        pltpu.VMEM((NCH, CH), jnp.int32),
        pltpu.VMEM((CH,), jnp.float32),
        pltpu.VMEM_SHARED((NPAD,), jnp.float32),
    ],
)
def _deg_kernel(dst_hbm, ones_hbm, zeros_hbm, degp_hbm, dstv, onesv, deg_sp):
    c = lax.axis_index("c")
    s = lax.axis_index("s")
    w = c * NS + s
    pltpu.sync_copy(dst_hbm.at[w], dstv)
    pltpu.sync_copy(ones_hbm, onesv)

    @pl.when(s == 0)
    def _():
        pltpu.sync_copy(zeros_hbm, deg_sp)

    plsc.subcore_barrier()

    def step(j, carry):
        pltpu.sync_copy(onesv, deg_sp.at[dstv.at[j]], add=True)
        return carry

    lax.fori_loop(0, NCH, step, 0)
    plsc.subcore_barrier()
    rps = NPAD // NS
    pltpu.sync_copy(deg_sp.at[pl.ds(s * rps, rps)], degp_hbm.at[c, pl.ds(s * rps, rps)])


def _make_edge_pass(F):
    scratch = [
        pltpu.VMEM((NCH, CH), jnp.int32),
        pltpu.VMEM((NCH, CH), jnp.int32),
    ]
    scratch += [pltpu.VMEM((CH,), jnp.float32) for _ in range(F)]
    scratch += [pltpu.VMEM_SHARED((NPAD,), jnp.float32) for _ in range(2 * F)]
    scratch += [pltpu.SemaphoreType.DMA]

    @functools.partial(
        pl.kernel,
        out_type=[jax.ShapeDtypeStruct((NPAD,), jnp.float32)] * (NC * F),
        mesh=_MESH,
        scratch_types=scratch,
    )
    def _edge_kernel(src_hbm, dst_hbm, *rest):
        g = rest[0:F]
        zeros_hbm = rest[F]
        outs = rest[F + 1:F + 1 + NC * F]
        scr = rest[F + 1 + NC * F:]
        srcv, dstv = scr[0], scr[1]
        rows = scr[2:2 + F]
        tbl = scr[2 + F:2 + 2 * F]
        acc = scr[2 + 2 * F:2 + 3 * F]
        sem = scr[-1]
        c = lax.axis_index("c")
        s = lax.axis_index("s")
        w = c * NS + s
        pltpu.sync_copy(src_hbm.at[w], srcv)
        pltpu.sync_copy(dst_hbm.at[w], dstv)

        for f in range(F):
            @pl.when(s == f)
            def _(f=f):
                pltpu.sync_copy(g[f], tbl[f])

            @pl.when(jnp.logical_and(s == F + f, c == 0))
            def _(f=f):
                pltpu.sync_copy(g[f], acc[f])

            @pl.when(jnp.logical_and(s == F + f, c == 1))
            def _(f=f):
                pltpu.sync_copy(zeros_hbm, acc[f])

        plsc.subcore_barrier()

        def step(j, carry):
            sidx = srcv.at[j]
            didx = dstv.at[j]
            cps = [pltpu.async_copy(tbl[f].at[sidx], rows[f], sem)
                   for f in range(F)]
            for cp in cps:
                cp.wait()
            cps = [pltpu.async_copy(rows[f], acc[f].at[didx], sem,
                                    add=True) for f in range(F)]
            for cp in cps:
                cp.wait()
            return carry

        lax.fori_loop(0, NCH, step, 0)
        plsc.subcore_barrier()

        for cc in range(NC):
            for f in range(F):
                @pl.when(jnp.logical_and(c == cc, s == f))
                def _(cc=cc, f=f):
                    pltpu.sync_copy(acc[f], outs[cc * F + f])

    return _edge_kernel


_edge_pass_8 = _make_edge_pass(8)

_edge_pass_2 = _make_edge_pass(2)


_DN = (((0,), (1,)), ((), ()))
_DF = (((0,), (0,)), ((), ()))


def _tc_mm_body(x_ref, w1_ref, h_ref):
    h_ref[...] = lax.dot_general(w1_ref[...], x_ref[...], _DN,
                                 preferred_element_type=jnp.float32)


def _tc_scale_body(h_ref, degp_ref, g_ref, dis_ref):
    dp = degp_ref[...]
    deg = dp[0:1, :] + dp[1:2, :] + 1.0
    dis = lax.rsqrt(deg)
    g_ref[...] = dis * h_ref[...]
    dis_ref[...] = dis


def _tc_mid_body(accp_ref, dis_ref, b1_ref, w2_ref, g2_ref):
    ap = accp_ref[...]
    dis = dis_ref[...]
    h = jnp.tanh(dis * (ap[0] + ap[1]) + b1_ref[...])
    g2_ref[...] = dis * lax.dot_general(w2_ref[...], h, _DF,
                                        preferred_element_type=jnp.float32)


def _tc_post_body(accp_ref, dis_ref, b2_ref, wc_ref, bc_ref, out_ref):
    ap = accp_ref[...]
    emb = jnp.tanh(dis_ref[...] * (ap[0] + ap[1]) + b2_ref[...])
    z = lax.dot_general(wc_ref[...], emb, _DF,
                        preferred_element_type=jnp.float32) + bc_ref[...]
    out_ref[...] = jax.nn.sigmoid(z)


_tc_mm = pl.pallas_call(
    _tc_mm_body,
    out_shape=jax.ShapeDtypeStruct((8, NPAD), jnp.float32),
)

_tc_scale = pl.pallas_call(
    _tc_scale_body,
    out_shape=(
        jax.ShapeDtypeStruct((8, NPAD), jnp.float32),
        jax.ShapeDtypeStruct((1, NPAD), jnp.float32),
    ),
)

_tc_mid = pl.pallas_call(
    _tc_mid_body,
    out_shape=jax.ShapeDtypeStruct((2, NPAD), jnp.float32),
)

_tc_post = pl.pallas_call(
    _tc_post_body,
    out_shape=jax.ShapeDtypeStruct((1, NPAD), jnp.float32),
)


def kernel(x, edge_index, W1, b1, W2, b2, Wc, bc):
    ei = edge_index.astype(jnp.int32)
    pad = jnp.full((EPAD - E,), NPAD - 1, jnp.int32)
    srcf = jnp.concatenate([ei[0], pad])
    dstf = jnp.concatenate([ei[1], pad])
    srcr = srcf.reshape(NW, NCH, CH)
    dstr = dstf.reshape(NW, NCH, CH)
    x_pad = jnp.pad(x, ((0, NPAD - N), (0, 0)))
    ones_ch = jnp.ones((CH,), jnp.float32)
    zeros1 = jnp.zeros((NPAD,), jnp.float32)

    h1 = _tc_mm(x_pad, W1)
    degp = _deg_kernel(dstr, ones_ch, zeros1)
    g1, dis = _tc_scale(h1, degp)
    outs1 = _edge_pass_8(srcr, dstr, *[g1[f] for f in range(8)], zeros1)
    accp1 = jnp.stack(outs1).reshape(NC, 8, NPAD)
    g2 = _tc_mid(accp1, dis, b1.reshape(8, 1), W2)
    outs2 = _edge_pass_2(srcr, dstr, *[g2[f] for f in range(2)], zeros1)
    accp2 = jnp.stack(outs2).reshape(NC, 2, NPAD)
    out = _tc_post(accp2, dis, b2.reshape(2, 1), Wc, bc.reshape(1, 1))
    return out.reshape(NPAD, 1)[:N]
